# baseline (speedup 1.0000x reference)
import functools

import jax
import jax.numpy as jnp
from jax import lax
from jax.experimental import pallas as pl
from jax.experimental.pallas import tpu as pltpu
from jax.experimental.pallas import tpu_sc as plsc

BETA = 0.25


def _argmin_body(nk, bk, x_ref, e_ref, idx_ref, minv_ref, mini_ref):
    k = pl.program_id(1)
    x = x_ref[...]
    e = e_ref[...]
    cross = lax.dot_general(
        x, e, (((1,), (1,)), ((), ())),
        preferred_element_type=jnp.float32)
    x_sq = jnp.sum(x * x, axis=1, keepdims=True)
    e_sq = jnp.sum(e * e, axis=1)
    dist = (x_sq + e_sq[None, :]) - 2.0 * cross
    m = jnp.min(dist, axis=1, keepdims=True)
    cols = lax.broadcasted_iota(jnp.int32, dist.shape, 1)
    first = jnp.min(jnp.where(dist == m, cols, bk), axis=1, keepdims=True)
    gidx = first + k * bk

    @pl.when(k == 0)
    def _():
        minv_ref[...] = m
        mini_ref[...] = gidx

    @pl.when(k > 0)
    def _():
        better = m < minv_ref[...]
        minv_ref[...] = jnp.where(better, m, minv_ref[...])
        mini_ref[...] = jnp.where(better, gidx, mini_ref[...])

    @pl.when(k == nk - 1)
    def _():
        idx_ref[0, 0, :] = mini_ref[...][:, 0]


def _nearest_code(x_flat, embed_weight, bm=512, bk=512):
    m, d = x_flat.shape
    kk, _ = embed_weight.shape
    nm, nk = m // bm, kk // bk
    idx3 = pl.pallas_call(
        functools.partial(_argmin_body, nk, bk),
        grid=(nm, nk),
        in_specs=[
            pl.BlockSpec((bm, d), lambda i, k: (i, 0)),
            pl.BlockSpec((bk, d), lambda i, k: (k, 0)),
        ],
        out_specs=pl.BlockSpec((1, 1, bm), lambda i, k: (i, 0, 0)),
        out_shape=jax.ShapeDtypeStruct((nm, 1, bm), jnp.int32),
        scratch_shapes=[
            pltpu.VMEM((bm, 1), jnp.float32),
            pltpu.VMEM((bm, 1), jnp.int32),
        ],
        compiler_params=pltpu.CompilerParams(
            dimension_semantics=("arbitrary", "arbitrary")),
    )(x_flat, embed_weight)
    return idx3.reshape(m)


def _gather_rows(embed_weight, inds):
    kk, d = embed_weight.shape
    m = inds.shape[0]
    info = plsc.get_sparse_core_info()
    nc, ns = info.num_cores, info.num_subcores
    nw = nc * ns
    b_per_w = m // nw
    nchunk = b_per_w // 128
    idx2 = inds.reshape(m // 128, 128)
    mesh = plsc.VectorSubcoreMesh(core_axis_name="c", subcore_axis_name="s")

    @functools.partial(
        pl.kernel, mesh=mesh,
        out_type=jax.ShapeDtypeStruct((m, d), jnp.float32),
        scratch_types=[
            pltpu.VMEM((nchunk, 128), jnp.int32),
            pltpu.VMEM((b_per_w, d), jnp.float32),
            pltpu.SemaphoreType.DMA,
        ],
    )
    def gather_kernel(table_hbm, idx_hbm, out_hbm, idx_v, rows_v, sem):
        wid = lax.axis_index("s") * nc + lax.axis_index("c")
        pltpu.sync_copy(idx_hbm.at[pl.ds(wid * nchunk, nchunk)], idx_v)
        copies = []
        for j in range(nchunk):
            copies.append(pltpu.async_copy(
                table_hbm.at[idx_v.at[j]],
                rows_v.at[pl.ds(j * 128, 128)], sem))
        for cp in copies:
            cp.wait()
        pltpu.sync_copy(rows_v, out_hbm.at[pl.ds(wid * b_per_w, b_per_w)])

    return gather_kernel(embed_weight, idx2)


def _assemble_body(nb, inv_n, xq_ref, x_ref, out_ref, loss_ref, acc_ref):
    b = pl.program_id(0)
    xq_t = xq_ref[0].T
    x = x_ref[0]
    out_ref[0] = x + (xq_t - x)
    t = xq_t - x
    t2 = t * t
    v = t2 * BETA + t2
    s = jnp.sum(v)

    @pl.when(b == 0)
    def _():
        acc_ref[0, 0] = s

    @pl.when(b > 0)
    def _():
        acc_ref[0, 0] = acc_ref[0, 0] + s

    @pl.when(b == nb - 1)
    def _():
        loss_ref[...] = jnp.broadcast_to(acc_ref[0, 0] * inv_n, (1, 1))


def _assemble(xq_rows, x_lat3):
    b, c, hw = x_lat3.shape
    xq3 = xq_rows.reshape(b, hw, c)
    n = b * c * hw
    out3, loss = pl.pallas_call(
        functools.partial(_assemble_body, b, 1.0 / n),
        grid=(b,),
        in_specs=[
            pl.BlockSpec((1, hw, c), lambda i: (i, 0, 0)),
            pl.BlockSpec((1, c, hw), lambda i: (i, 0, 0)),
        ],
        out_specs=[
            pl.BlockSpec((1, c, hw), lambda i: (i, 0, 0)),
            pl.BlockSpec((1, 1), lambda i: (0, 0)),
        ],
        out_shape=[
            jax.ShapeDtypeStruct((b, c, hw), jnp.float32),
            jax.ShapeDtypeStruct((1, 1), jnp.float32),
        ],
        scratch_shapes=[pltpu.SMEM((1, 1), jnp.float32)],
        compiler_params=pltpu.CompilerParams(
            dimension_semantics=("arbitrary",)),
    )(xq3, x_lat3)
    return out3, loss[0, 0]


def kernel(x_latent, embed_weight):
    b, c, h, w = x_latent.shape
    x_lat3 = x_latent.reshape(b, c, h * w)
    x_flat = jnp.transpose(x_lat3, (0, 2, 1)).reshape(b * h * w, c)
    inds = _nearest_code(x_flat, embed_weight)
    xq_rows = _gather_rows(embed_weight, inds)
    out3, loss = _assemble(xq_rows, x_lat3)
    return out3.reshape(b, c, h, w), loss


# speedup vs baseline: 1.4191x; 1.4191x over previous
import functools

import jax
import jax.numpy as jnp
from jax import lax
from jax.experimental import pallas as pl
from jax.experimental.pallas import tpu as pltpu
from jax.experimental.pallas import tpu_sc as plsc

BETA = 0.25


def _tree_min(a):
    n = a.shape[0]
    while n > 8:
        half = n // 2
        a = jnp.minimum(a[:half], a[half:])
        n = half
    return jnp.min(a, axis=0, keepdims=True)


def _argmin_body(nk, bk, x_ref, e_ref, idx_ref, minv_ref, mini_ref, xsq_ref,
                 esq_ref, rows_ref):
    i = pl.program_id(0)
    k = pl.program_id(1)
    x = x_ref[...]
    e = e_ref[...]

    @pl.when(k == 0)
    def _():
        ones = jnp.ones((1, x.shape[1]), jnp.float32)
        xsq_ref[...] = lax.dot_general(
            ones, x * x, (((1,), (1,)), ((), ())),
            preferred_element_type=jnp.float32)

    @pl.when(i == 0)
    def _():
        esq_ref[pl.ds(k * bk, bk), :] = jnp.sum(e * e, axis=1, keepdims=True)

    @pl.when((i == 0) & (k == 0))
    def _():
        rows_ref[...] = lax.broadcasted_iota(
            jnp.int32, rows_ref.shape, 0).astype(jnp.float32)

    cross_t = lax.dot_general(
        e, x, (((1,), (1,)), ((), ())),
        preferred_element_type=jnp.float32)
    e_sq = esq_ref[pl.ds(k * bk, bk), :]
    dist = (xsq_ref[...] + e_sq) - 2.0 * cross_t
    m = _tree_min(dist)
    first = _tree_min(jnp.where(dist == m, rows_ref[...], float(bk)))
    gidx = first.astype(jnp.int32) + k * bk

    @pl.when(k == 0)
    def _():
        minv_ref[...] = m
        mini_ref[...] = gidx

    @pl.when(k > 0)
    def _():
        better = m < minv_ref[...]
        minv_ref[...] = jnp.where(better, m, minv_ref[...])
        mini_ref[...] = jnp.where(better, gidx, mini_ref[...])

    @pl.when(k == nk - 1)
    def _():
        idx_ref[0, 0, :] = mini_ref[0, :]


def _nearest_code(x_flat, embed_weight, bm=512, bk=1024):
    m, d = x_flat.shape
    kk, _ = embed_weight.shape
    nm, nk = m // bm, kk // bk
    idx3 = pl.pallas_call(
        functools.partial(_argmin_body, nk, bk),
        grid=(nm, nk),
        in_specs=[
            pl.BlockSpec((bm, d), lambda i, k: (i, 0)),
            pl.BlockSpec((bk, d), lambda i, k: (k, 0)),
        ],
        out_specs=pl.BlockSpec((1, 1, bm), lambda i, k: (i, 0, 0)),
        out_shape=jax.ShapeDtypeStruct((nm, 1, bm), jnp.int32),
        scratch_shapes=[
            pltpu.VMEM((1, bm), jnp.float32),
            pltpu.VMEM((1, bm), jnp.int32),
            pltpu.VMEM((1, bm), jnp.float32),
            pltpu.VMEM((kk, 1), jnp.float32),
            pltpu.VMEM((bk, bm), jnp.float32),
        ],
        compiler_params=pltpu.CompilerParams(
            dimension_semantics=("arbitrary", "arbitrary")),
    )(x_flat, embed_weight)
    return idx3.reshape(m)


def _gather_rows(embed_weight, inds):
    kk, d = embed_weight.shape
    m = inds.shape[0]
    info = plsc.get_sparse_core_info()
    nc, ns = info.num_cores, info.num_subcores
    nw = nc * ns
    b_per_w = m // nw
    nchunk = b_per_w // 128
    idx2 = inds.reshape(m // 128, 128)
    mesh = plsc.VectorSubcoreMesh(core_axis_name="c", subcore_axis_name="s")

    @functools.partial(
        pl.kernel, mesh=mesh,
        out_type=jax.ShapeDtypeStruct((m, d), jnp.float32),
        scratch_types=[
            pltpu.VMEM((nchunk, 128), jnp.int32),
            pltpu.VMEM((b_per_w, d), jnp.float32),
            pltpu.SemaphoreType.DMA,
        ],
    )
    def gather_kernel(table_hbm, idx_hbm, out_hbm, idx_v, rows_v, sem):
        wid = lax.axis_index("s") * nc + lax.axis_index("c")
        pltpu.sync_copy(idx_hbm.at[pl.ds(wid * nchunk, nchunk)], idx_v)
        copies = []
        for j in range(nchunk):
            copies.append(pltpu.async_copy(
                table_hbm.at[idx_v.at[j]],
                rows_v.at[pl.ds(j * 128, 128)], sem))
        for cp in copies:
            cp.wait()
        pltpu.sync_copy(rows_v, out_hbm.at[pl.ds(wid * b_per_w, b_per_w)])

    return gather_kernel(embed_weight, idx2)


def _assemble_body(nb, inv_n, xq_ref, x_ref, out_ref, loss_ref, acc_ref):
    b = pl.program_id(0)
    xq_t = xq_ref[0].T
    x = x_ref[0]
    out_ref[0] = x + (xq_t - x)
    t = xq_t - x
    t2 = t * t
    v = t2 * BETA + t2
    s = jnp.sum(v)

    @pl.when(b == 0)
    def _():
        acc_ref[0, 0] = s

    @pl.when(b > 0)
    def _():
        acc_ref[0, 0] = acc_ref[0, 0] + s

    @pl.when(b == nb - 1)
    def _():
        loss_ref[...] = jnp.broadcast_to(acc_ref[0, 0] * inv_n, (1, 1))


def _assemble(xq_rows, x_lat3):
    b, c, hw = x_lat3.shape
    xq3 = xq_rows.reshape(b, hw, c)
    n = b * c * hw
    out3, loss = pl.pallas_call(
        functools.partial(_assemble_body, b, 1.0 / n),
        grid=(b,),
        in_specs=[
            pl.BlockSpec((1, hw, c), lambda i: (i, 0, 0)),
            pl.BlockSpec((1, c, hw), lambda i: (i, 0, 0)),
        ],
        out_specs=[
            pl.BlockSpec((1, c, hw), lambda i: (i, 0, 0)),
            pl.BlockSpec((1, 1), lambda i: (0, 0)),
        ],
        out_shape=[
            jax.ShapeDtypeStruct((b, c, hw), jnp.float32),
            jax.ShapeDtypeStruct((1, 1), jnp.float32),
        ],
        scratch_shapes=[pltpu.SMEM((1, 1), jnp.float32)],
        compiler_params=pltpu.CompilerParams(
            dimension_semantics=("arbitrary",)),
    )(xq3, x_lat3)
    return out3, loss[0, 0]


def kernel(x_latent, embed_weight):
    b, c, h, w = x_latent.shape
    x_lat3 = x_latent.reshape(b, c, h * w)
    x_flat = jnp.transpose(x_lat3, (0, 2, 1)).reshape(b * h * w, c)
    inds = _nearest_code(x_flat, embed_weight)
    xq_rows = _gather_rows(embed_weight, inds)
    out3, loss = _assemble(xq_rows, x_lat3)
    return out3.reshape(b, c, h, w), loss
